# Initial kernel scaffold; baseline (speedup 1.0000x reference)
import jax
import jax.numpy as jnp
from jax.experimental import pallas as pl


def kernel(input_ids, pos_ids, word_table, pos_table):
    raise NotImplementedError("write your pallas kernel here")


# speedup vs baseline: 4.3004x; 4.3004x over previous
import functools

import jax
import jax.numpy as jnp
from jax import lax
from jax.experimental import pallas as pl
from jax.experimental.pallas import tpu as pltpu
from jax.experimental.pallas import tpu_sc as plsc

NC = 2
NS = 16
NW = NC * NS

B = 4096 * 200
D = 64
PER_W = B // NW
K = 128
N_CHUNKS = PER_W // K


def _body(widx_hbm, pidx_hbm, word_hbm, pos_hbm, out_hbm,
          widx_v, pidx_v, acc_v, sem):
    wid = lax.axis_index("s") * NC + lax.axis_index("c")
    w_base = wid * PER_W

    def chunk(i, carry):
        base = w_base + i * K
        pltpu.sync_copy(widx_hbm.at[pl.ds(base, K)], widx_v)
        pltpu.sync_copy(pidx_hbm.at[pl.ds(base, K)], pidx_v)
        pltpu.async_copy(pos_hbm.at[pidx_v], acc_v, sem).wait()
        pltpu.async_copy(word_hbm.at[widx_v], acc_v, sem, add=True).wait()
        pltpu.sync_copy(acc_v, out_hbm.at[pl.ds(base, K)])
        return carry

    lax.fori_loop(0, N_CHUNKS, chunk, 0)


@jax.jit
def _emb(widx, pidx, word_table, pos_table):
    mesh = plsc.VectorSubcoreMesh(
        core_axis_name="c", subcore_axis_name="s",
        num_cores=NC, num_subcores=NS)
    f = pl.kernel(
        _body,
        out_type=jax.ShapeDtypeStruct((B, D), jnp.float32),
        mesh=mesh,
        compiler_params=pltpu.CompilerParams(use_tc_tiling_on_sc=False),
        scratch_types=[
            pltpu.VMEM((K,), jnp.int32),
            pltpu.VMEM((K,), jnp.int32),
            pltpu.VMEM((K, D), jnp.float32),
            pltpu.SemaphoreType.DMA,
        ],
    )
    return f(widx, pidx, word_table, pos_table)


def kernel(input_ids, pos_ids, word_table, pos_table):
    widx = input_ids.reshape(-1).astype(jnp.int32)
    pidx = pos_ids.reshape(-1).astype(jnp.int32)
    out = _emb(widx, pidx, word_table, pos_table)
    return out.reshape(input_ids.shape + (D,))


# speedup vs baseline: 4.6595x; 1.0835x over previous
import jax
import jax.numpy as jnp
from jax import lax
from jax.experimental import pallas as pl
from jax.experimental.pallas import tpu as pltpu
from jax.experimental.pallas import tpu_sc as plsc

NC = 2
NS = 16
NW = NC * NS

B = 4096 * 200
D = 64
PER_W = B // NW
K = 128
N_CHUNKS = PER_W // K
NBUF = 4
N_SLOTS = N_CHUNKS + NBUF
N_OUTER = (N_SLOTS + NBUF - 1) // NBUF


def _body(widx_hbm, pidx_hbm, word_hbm, pos_hbm, out_hbm,
          widx_v, pidx_v, acc_v, sem_idx, sem_pos, sem_word, sem_out):
    wid = lax.axis_index("s") * NC + lax.axis_index("c")
    w_base = wid * PER_W

    def fire_idx(i, b):
        base = w_base + i * K
        pltpu.async_copy(widx_hbm.at[pl.ds(base, K)], widx_v.at[b], sem_idx.at[b])
        pltpu.async_copy(pidx_hbm.at[pl.ds(base, K)], pidx_v.at[b], sem_idx.at[b])

    def wait_idx(i, b):
        base = w_base + i * K
        pltpu.make_async_copy(widx_hbm.at[pl.ds(base, K)], widx_v.at[b], sem_idx.at[b]).wait()
        pltpu.make_async_copy(pidx_hbm.at[pl.ds(base, K)], pidx_v.at[b], sem_idx.at[b]).wait()

    def fire_pos(b):
        pltpu.async_copy(pos_hbm.at[pidx_v.at[b]], acc_v.at[b], sem_pos.at[b])

    def wait_pos(b):
        pltpu.make_async_copy(pos_hbm.at[pidx_v.at[b]], acc_v.at[b], sem_pos.at[b]).wait()

    def fire_word(b):
        pltpu.async_copy(word_hbm.at[widx_v.at[b]], acc_v.at[b], sem_word.at[b], add=True)

    def wait_word(b):
        pltpu.make_async_copy(word_hbm.at[widx_v.at[b]], acc_v.at[b], sem_word.at[b]).wait()

    def fire_out(i, b):
        base = w_base + i * K
        pltpu.async_copy(acc_v.at[b], out_hbm.at[pl.ds(base, K)], sem_out.at[b])

    def wait_out(i, b):
        base = w_base + i * K
        pltpu.make_async_copy(acc_v.at[b], out_hbm.at[pl.ds(base, K)], sem_out.at[b]).wait()

    def outer(it, carry):
        g0 = it * NBUF
        for b in range(NBUF):
            slot = g0 + b
            i0 = slot - NBUF

            @pl.when(slot >= NBUF)
            def _():
                wait_out(i0, b)

            @pl.when(slot < N_CHUNKS)
            def _():
                fire_idx(slot, b)

            j = slot - 1
            b1 = (b - 1) % NBUF

            @pl.when(jnp.logical_and(j >= 0, j < N_CHUNKS))
            def _():
                wait_idx(j, b1)
                fire_pos(b1)

            k2 = slot - 2
            b2 = (b - 2) % NBUF

            @pl.when(jnp.logical_and(k2 >= 0, k2 < N_CHUNKS))
            def _():
                wait_pos(b2)
                fire_word(b2)

            m = slot - 3
            b3 = (b - 3) % NBUF

            @pl.when(jnp.logical_and(m >= 0, m < N_CHUNKS))
            def _():
                wait_word(b3)
                fire_out(m, b3)

        return carry

    lax.fori_loop(0, N_OUTER, outer, 0)


@jax.jit
def _emb(widx, pidx, word_table, pos_table):
    mesh = plsc.VectorSubcoreMesh(
        core_axis_name="c", subcore_axis_name="s",
        num_cores=NC, num_subcores=NS)
    f = pl.kernel(
        _body,
        out_type=jax.ShapeDtypeStruct((B, D), jnp.float32),
        mesh=mesh,
        compiler_params=pltpu.CompilerParams(use_tc_tiling_on_sc=False),
        scratch_types=[
            pltpu.VMEM((NBUF, K), jnp.int32),
            pltpu.VMEM((NBUF, K), jnp.int32),
            pltpu.VMEM((NBUF, K, D), jnp.float32),
            pltpu.SemaphoreType.DMA((NBUF,)),
            pltpu.SemaphoreType.DMA((NBUF,)),
            pltpu.SemaphoreType.DMA((NBUF,)),
            pltpu.SemaphoreType.DMA((NBUF,)),
        ],
    )
    return f(widx, pidx, word_table, pos_table)


def kernel(input_ids, pos_ids, word_table, pos_table):
    widx = input_ids.reshape(-1).astype(jnp.int32)
    pidx = pos_ids.reshape(-1).astype(jnp.int32)
    out = _emb(widx, pidx, word_table, pos_table)
    return out.reshape(input_ids.shape + (D,))
